# baseline (speedup 1.0000x reference)
import functools

import jax
import jax.numpy as jnp
from jax import lax
from jax.experimental import pallas as pl
from jax.experimental.pallas import tpu as pltpu
from jax.experimental.pallas import tpu_sc as plsc

_VOCAB = 1000000
_EMB = 32
_B = 16384
_L = 200

_LANES = 128
_PACK = _LANES // _EMB
_TVROWS = _VOCAB // _PACK
_TV_BLK = 2000

_NC, _NS, _SL = 2, 16, 16
_NW = _NC * _NS
_R = 64
_G = _B // _R
_GPW = _G // _NW
_IDXN = _R * _L
_IDX_ROWS = _IDXN // _LANES
_RPW = _B // _NW


def _tv_body(resh_ref, w4_ref, b_ref, out_ref):
    out_ref[...] = (
        jnp.dot(resh_ref[...], w4_ref[...], preferred_element_type=jnp.float32)
        + b_ref[0, 0]
    )


def _compute_tv(resh, w4, b2):
    return pl.pallas_call(
        _tv_body,
        grid=(_TVROWS // _TV_BLK,),
        in_specs=[
            pl.BlockSpec((_TV_BLK, _LANES), lambda i: (i, 0)),
            pl.BlockSpec((_LANES, _PACK), lambda i: (0, 0)),
            pl.BlockSpec(memory_space=pltpu.SMEM),
        ],
        out_specs=pl.BlockSpec((_TV_BLK, _PACK), lambda i: (i, 0)),
        out_shape=jax.ShapeDtypeStruct((_TVROWS, _PACK), jnp.float32),
    )(resh, w4, b2)


_UNROLL = 4


def _pool_body(tv_hbm, xg_hbm, out_hbm, idx_a, idx_b, val_a, val_b, drain_v,
               out_v, sem_a, sem_b):
    wid = lax.axis_index("s") * _NC + lax.axis_index("c")
    g0 = wid * _GPW
    idxs, vals, sems = (idx_a, idx_b), (val_a, val_b), (sem_a, sem_b)
    row_base = lax.iota(jnp.int32, _SL) * _L

    def stage(j):
        b = j & 1
        pltpu.sync_copy(
            xg_hbm.at[pl.ds((g0 + j) * _IDXN, _IDXN)], idxs[b]
        )

        def fire(c, u):
            pltpu.async_copy(
                tv_hbm.at[idxs[b].at[pl.ds(c * _LANES, _LANES)]],
                vals[b].at[c],
                sems[b],
            )
            return u

        lax.fori_loop(0, _IDX_ROWS, fire, 0)

    def drain_reduce(j):
        b = j & 1
        pltpu.make_async_copy(
            tv_hbm.at[pl.ds(0, _IDXN)], drain_v, sems[b]
        ).wait()
        for q in range(_R // _SL):
            base = row_base + q * _SL * _L

            def red(t, acc):
                o = base + t * _UNROLL
                for u in range(_UNROLL):
                    ou = o + u
                    acc = acc + plsc.load_gather(
                        vals[b], [ou >> 7, ou & 127]
                    )
                return acc

            acc = lax.fori_loop(0, _L // _UNROLL, red,
                                jnp.zeros((_SL,), jnp.float32))
            out_v[pl.ds(j * _R + q * _SL, _SL)] = acc

    stage(0)
    for j in range(_GPW):
        if j + 1 < _GPW:
            stage(j + 1)
        drain_reduce(j)
    pltpu.sync_copy(out_v, out_hbm.at[pl.ds(wid * _RPW, _RPW)])


def _pool_call(tv, xg):
    run = pl.kernel(
        _pool_body,
        mesh=plsc.VectorSubcoreMesh(core_axis_name="c", subcore_axis_name="s"),
        compiler_params=pltpu.CompilerParams(needs_layout_passes=False),
        out_type=jax.ShapeDtypeStruct((_B,), jnp.float32),
        scratch_types=[
            pltpu.VMEM((_IDXN,), jnp.int32),
            pltpu.VMEM((_IDXN,), jnp.int32),
            pltpu.VMEM((_IDX_ROWS, _LANES), jnp.float32),
            pltpu.VMEM((_IDX_ROWS, _LANES), jnp.float32),
            pltpu.VMEM((_IDXN,), jnp.float32),
            pltpu.VMEM((_RPW,), jnp.float32),
            pltpu.SemaphoreType.DMA,
            pltpu.SemaphoreType.DMA,
        ],
    )
    return run(tv, xg)


def kernel(x, table, W, b):
    xi = x.astype(jnp.int32)
    resh = table.reshape(_TVROWS, _LANES)
    w4 = jnp.kron(jnp.eye(_PACK, dtype=jnp.float32), W * (1.0 / _L))
    b2 = (b * (1.0 / _L)).reshape(1, 1).astype(jnp.float32)
    tv = _compute_tv(resh, w4, b2).reshape(_VOCAB)
    xg = xi.reshape(_B * _L)
    out = _pool_call(tv, xg)
    return out.reshape(_B, 1)


# speedup vs baseline: 3.6748x; 3.6748x over previous
import functools

import jax
import jax.numpy as jnp
from jax import lax
from jax.experimental import pallas as pl
from jax.experimental.pallas import tpu as pltpu
from jax.experimental.pallas import tpu_sc as plsc

_VOCAB = 1000000
_EMB = 32
_B = 16384
_L = 200

_LANES = 128
_TV_BLK = 8192

_NC, _NS, _SL = 2, 16, 16
_NW = _NC * _NS
_RPW = _B // _NW
_GPW = _RPW // _LANES
_ACCS = _LANES // _SL


def _tv_body(t_ref, w_ref, b_ref, o_ref):
    o_ref[...] = jnp.sum(t_ref[...] * w_ref[...], axis=0) + b_ref[0, 0]


def _compute_tv(tT, wl, bl):
    grid = (_VOCAB + _TV_BLK - 1) // _TV_BLK
    return pl.pallas_call(
        _tv_body,
        grid=(grid,),
        in_specs=[
            pl.BlockSpec((_EMB, _TV_BLK), lambda i: (0, i)),
            pl.BlockSpec((_EMB, 1), lambda i: (0, 0)),
            pl.BlockSpec(memory_space=pltpu.SMEM),
        ],
        out_specs=pl.BlockSpec((_TV_BLK,), lambda i: (i,)),
        out_shape=jax.ShapeDtypeStruct((_VOCAB,), jnp.float32),
    )(tT, wl, bl)


def _pool_body(tv_hbm, xT_hbm, out_hbm, idx_a, idx_b, val_a, val_b, out_v,
               sem_a, sem_b):
    wid = lax.axis_index("s") * _NC + lax.axis_index("c")
    col0 = wid * _RPW
    idxs, vals, sems = (idx_a, idx_b), (val_a, val_b), (sem_a, sem_b)

    def stage(j):
        b = j & 1
        pltpu.sync_copy(
            xT_hbm.at[:, pl.ds(col0 + j * _LANES, _LANES)], idxs[b]
        )

        def fire(l, u):
            pltpu.async_copy(
                tv_hbm.at[idxs[b].at[l]], vals[b].at[l], sems[b]
            )
            return u

        lax.fori_loop(0, _L, fire, 0)

    def drain_reduce(j):
        b = j & 1

        def drain(l, u):
            pltpu.make_async_copy(
                tv_hbm.at[pl.ds(0, _LANES)], vals[b].at[l], sems[b]
            ).wait()
            return u

        lax.fori_loop(0, _L, drain, 0)

        def red(l, accs):
            return tuple(
                accs[k] + vals[b][l, pl.ds(k * _SL, _SL)]
                for k in range(_ACCS)
            )

        zero = jnp.zeros((_SL,), jnp.float32)
        accs = lax.fori_loop(0, _L, red, (zero,) * _ACCS)
        for k in range(_ACCS):
            out_v[pl.ds(j * _LANES + k * _SL, _SL)] = accs[k]

    stage(0)
    for j in range(_GPW):
        if j + 1 < _GPW:
            stage(j + 1)
        drain_reduce(j)
    pltpu.sync_copy(out_v, out_hbm.at[pl.ds(col0, _RPW)])


def _pool_call(tv, xT):
    run = pl.kernel(
        _pool_body,
        mesh=plsc.VectorSubcoreMesh(core_axis_name="c", subcore_axis_name="s"),
        compiler_params=pltpu.CompilerParams(needs_layout_passes=False),
        out_type=jax.ShapeDtypeStruct((_B,), jnp.float32),
        scratch_types=[
            pltpu.VMEM((_L, _LANES), jnp.int32),
            pltpu.VMEM((_L, _LANES), jnp.int32),
            pltpu.VMEM((_L, _LANES), jnp.float32),
            pltpu.VMEM((_L, _LANES), jnp.float32),
            pltpu.VMEM((_RPW,), jnp.float32),
            pltpu.SemaphoreType.DMA,
            pltpu.SemaphoreType.DMA,
        ],
    )
    return run(tv, xT)


def kernel(x, table, W, b):
    xT = x.astype(jnp.int32).T
    tT = table.T
    wl = (W * (1.0 / _L)).reshape(_EMB, 1).astype(jnp.float32)
    bl = (b * (1.0 / _L)).reshape(1, 1).astype(jnp.float32)
    tv = _compute_tv(tT, wl, bl)
    out = _pool_call(tv, xT)
    return out.reshape(_B, 1)


# speedup vs baseline: 4.5357x; 1.2343x over previous
import functools

import jax
import jax.numpy as jnp
from jax import lax
from jax.experimental import pallas as pl
from jax.experimental.pallas import tpu as pltpu
from jax.experimental.pallas import tpu_sc as plsc

_VOCAB = 1000000
_EMB = 32
_B = 16384
_L = 200

_LANES = 128
_TV_BLK = 32768

_NC, _NS, _SL = 2, 16, 16
_NW = _NC * _NS
_RPW = _B // _NW
_GPW = _RPW // _LANES
_ACCS = _LANES // _SL


def _tv_body(t_ref, w_ref, b_ref, o_ref):
    o_ref[...] = jnp.sum(t_ref[...] * w_ref[...], axis=0) + b_ref[0, 0]


def _compute_tv(tT, wl, bl):
    grid = (_VOCAB + _TV_BLK - 1) // _TV_BLK
    return pl.pallas_call(
        _tv_body,
        grid=(grid,),
        in_specs=[
            pl.BlockSpec((_EMB, _TV_BLK), lambda i: (0, i)),
            pl.BlockSpec((_EMB, 1), lambda i: (0, 0)),
            pl.BlockSpec(memory_space=pltpu.SMEM),
        ],
        out_specs=pl.BlockSpec((_TV_BLK,), lambda i: (i,)),
        out_shape=jax.ShapeDtypeStruct((_VOCAB,), jnp.float32),
    )(tT, wl, bl)


def _pool_body(tv_hbm, xT_hbm, out_hbm, idx_a, idx_b, val_a, val_b, out_v,
               sem_a, sem_b):
    wid = lax.axis_index("s") * _NC + lax.axis_index("c")
    col0 = wid * _RPW
    idxs, vals, sems = (idx_a, idx_b), (val_a, val_b), (sem_a, sem_b)

    def stage(j):
        b = j & 1
        pltpu.sync_copy(
            xT_hbm.at[:, pl.ds(col0 + j * _LANES, _LANES)], idxs[b]
        )

        def fire(l, u):
            pltpu.async_copy(
                tv_hbm.at[idxs[b].at[l]], vals[b].at[l], sems[b]
            )
            return u

        lax.fori_loop(0, _L, fire, 0)

    def drain_reduce(j):
        b = j & 1

        def drain(l, u):
            pltpu.make_async_copy(
                tv_hbm.at[pl.ds(0, _LANES)], vals[b].at[l], sems[b]
            ).wait()
            return u

        lax.fori_loop(0, _L, drain, 0)

        def red(l, accs):
            return tuple(
                accs[k] + vals[b][l, pl.ds(k * _SL, _SL)]
                for k in range(_ACCS)
            )

        zero = jnp.zeros((_SL,), jnp.float32)
        accs = lax.fori_loop(0, _L, red, (zero,) * _ACCS)
        for k in range(_ACCS):
            out_v[pl.ds(j * _LANES + k * _SL, _SL)] = accs[k]

    stage(0)
    for j in range(_GPW):
        if j + 1 < _GPW:
            stage(j + 1)
        drain_reduce(j)
    pltpu.sync_copy(out_v, out_hbm.at[pl.ds(col0, _RPW)])


def _pool_call(tv, xT):
    run = pl.kernel(
        _pool_body,
        mesh=plsc.VectorSubcoreMesh(core_axis_name="c", subcore_axis_name="s"),
        compiler_params=pltpu.CompilerParams(needs_layout_passes=False),
        out_type=jax.ShapeDtypeStruct((_B,), jnp.float32),
        scratch_types=[
            pltpu.VMEM((_L, _LANES), jnp.int32),
            pltpu.VMEM((_L, _LANES), jnp.int32),
            pltpu.VMEM((_L, _LANES), jnp.float32),
            pltpu.VMEM((_L, _LANES), jnp.float32),
            pltpu.VMEM((_RPW,), jnp.float32),
            pltpu.SemaphoreType.DMA,
            pltpu.SemaphoreType.DMA,
        ],
    )
    return run(tv, xT)


def kernel(x, table, W, b):
    xT = x.astype(jnp.int32).T
    tT = table.T
    wl = (W * (1.0 / _L)).reshape(_EMB, 1).astype(jnp.float32)
    bl = (b * (1.0 / _L)).reshape(1, 1).astype(jnp.float32)
    tv = _compute_tv(tT, wl, bl)
    out = _pool_call(tv, xT)
    return out.reshape(_B, 1)


# speedup vs baseline: 4.6926x; 1.0346x over previous
import functools

import jax
import jax.numpy as jnp
from jax import lax
from jax.experimental import pallas as pl
from jax.experimental.pallas import tpu as pltpu
from jax.experimental.pallas import tpu_sc as plsc

_VOCAB = 1000000
_EMB = 32
_B = 16384
_L = 200

_LANES = 128
_TV_BLK = 65536

_NC, _NS, _SL = 2, 16, 16
_NW = _NC * _NS
_RPW = _B // _NW
_GPW = _RPW // _LANES
_ACCS = _LANES // _SL


def _tv_body(t_ref, w_ref, b_ref, o_ref):
    o_ref[...] = jnp.sum(t_ref[...] * w_ref[...], axis=0) + b_ref[0, 0]


def _compute_tv(tT, wl, bl):
    grid = (_VOCAB + _TV_BLK - 1) // _TV_BLK
    return pl.pallas_call(
        _tv_body,
        grid=(grid,),
        in_specs=[
            pl.BlockSpec((_EMB, _TV_BLK), lambda i: (0, i)),
            pl.BlockSpec((_EMB, 1), lambda i: (0, 0)),
            pl.BlockSpec(memory_space=pltpu.SMEM),
        ],
        out_specs=pl.BlockSpec((_TV_BLK,), lambda i: (i,)),
        out_shape=jax.ShapeDtypeStruct((_VOCAB,), jnp.float32),
    )(tT, wl, bl)


def _pool_body(tv_hbm, xT_hbm, out_hbm, idx_a, idx_b, val_a, val_b, out_v,
               sem_a, sem_b):
    wid = lax.axis_index("s") * _NC + lax.axis_index("c")
    col0 = wid * _RPW
    idxs, vals, sems = (idx_a, idx_b), (val_a, val_b), (sem_a, sem_b)

    def stage(j):
        b = j & 1
        pltpu.sync_copy(
            xT_hbm.at[:, pl.ds(col0 + j * _LANES, _LANES)], idxs[b]
        )

        def fire(l, u):
            pltpu.async_copy(
                tv_hbm.at[idxs[b].at[l]], vals[b].at[l], sems[b]
            )
            return u

        lax.fori_loop(0, _L, fire, 0)

    def drain_reduce(j):
        b = j & 1

        def drain(l, u):
            pltpu.make_async_copy(
                tv_hbm.at[pl.ds(0, _LANES)], vals[b].at[l], sems[b]
            ).wait()
            return u

        lax.fori_loop(0, _L, drain, 0)

        def red(l, accs):
            return tuple(
                accs[k] + vals[b][l, pl.ds(k * _SL, _SL)]
                for k in range(_ACCS)
            )

        zero = jnp.zeros((_SL,), jnp.float32)
        accs = lax.fori_loop(0, _L, red, (zero,) * _ACCS)
        for k in range(_ACCS):
            out_v[pl.ds(j * _LANES + k * _SL, _SL)] = accs[k]

    stage(0)
    for j in range(_GPW):
        if j + 1 < _GPW:
            stage(j + 1)
        drain_reduce(j)
    pltpu.sync_copy(out_v, out_hbm.at[pl.ds(col0, _RPW)])


def _pool_call(tv, xT):
    run = pl.kernel(
        _pool_body,
        mesh=plsc.VectorSubcoreMesh(core_axis_name="c", subcore_axis_name="s"),
        compiler_params=pltpu.CompilerParams(needs_layout_passes=False),
        out_type=jax.ShapeDtypeStruct((_B,), jnp.float32),
        scratch_types=[
            pltpu.VMEM((_L, _LANES), jnp.int32),
            pltpu.VMEM((_L, _LANES), jnp.int32),
            pltpu.VMEM((_L, _LANES), jnp.float32),
            pltpu.VMEM((_L, _LANES), jnp.float32),
            pltpu.VMEM((_RPW,), jnp.float32),
            pltpu.SemaphoreType.DMA,
            pltpu.SemaphoreType.DMA,
        ],
    )
    return run(tv, xT)


def kernel(x, table, W, b):
    xT = x.astype(jnp.int32).T
    tT = table.T
    wl = (W * (1.0 / _L)).reshape(_EMB, 1).astype(jnp.float32)
    bl = (b * (1.0 / _L)).reshape(1, 1).astype(jnp.float32)
    tv = _compute_tv(tT, wl, bl)
    out = _pool_call(tv, xT)
    return out.reshape(_B, 1)
